# baseline (speedup 1.0000x reference)
import functools

import jax
import jax.numpy as jnp
from jax import lax
from jax.experimental import pallas as pl
from jax.experimental.pallas import tpu as pltpu
from jax.experimental.pallas import tpu_sc as plsc

_LANES = 128


@functools.lru_cache(maxsize=None)
def _build(B, S, D, NC, NS):
    NW = NC * NS
    C = B // _LANES
    R = D // 8
    mesh = plsc.VectorSubcoreMesh(core_axis_name="c", subcore_axis_name="s")

    @functools.partial(
        pl.kernel,
        mesh=mesh,
        out_type=jax.ShapeDtypeStruct((S, R, C, 8, _LANES), jnp.float32),
        scratch_types=[
            pltpu.VMEM((S, _LANES), jnp.int32),
            pltpu.VMEM((_LANES, D), jnp.float32),
            pltpu.VMEM((_LANES, D), jnp.float32),
            pltpu.VMEM((D, _LANES), jnp.float32),
            pltpu.VMEM((D, _LANES), jnp.float32),
            pltpu.SemaphoreType.DMA,
            pltpu.SemaphoreType.DMA,
            pltpu.SemaphoreType.DMA,
            pltpu.SemaphoreType.DMA,
        ],
        compiler_params=pltpu.CompilerParams(
            use_tc_tiling_on_sc=False, needs_layout_passes=False
        ),
    )
    def k(x_hbm, table_hbm, out_hbm, idx_v, buf_a, buf_b, tb_a, tb_b,
          gs_a, gs_b, ws_a, ws_b):
        w = lax.axis_index("s") * NC + lax.axis_index("c")
        pltpu.sync_copy(x_hbm.at[w], idx_v)
        lane16 = jnp.arange(16, dtype=jnp.int32)

        def gather(s, buf, gs):
            pltpu.async_copy(table_hbm.at[idx_v.at[s]], buf, gs)

        def gather_wait(s, buf, gs):
            pltpu.make_async_copy(table_hbm.at[idx_v.at[s]], buf, gs).wait()

        def transpose(buf, tb):
            def tbody(j, carry):
                rows = lane16 + j * 16
                for d in range(D):
                    v = plsc.load_gather(
                        buf, [rows, jnp.full((16,), d, jnp.int32)]
                    )
                    tb[d, pl.ds(j * 16, 16)] = v
                return carry

            lax.fori_loop(0, _LANES // 16, tbody, 0)

        def writes(s, tb, ws):
            for r in range(R):
                pltpu.async_copy(
                    tb.at[pl.ds(8 * r, 8)], out_hbm.at[s, r, w], ws
                )

        def writes_wait(s, tb, ws):
            for r in range(R):
                pltpu.make_async_copy(
                    tb.at[pl.ds(8 * r, 8)], out_hbm.at[s, r, w], ws
                ).wait()

        gather(0, buf_a, gs_a)

        def step(s, buf, tb, gs, ws, obuf, ogs):
            @pl.when(s + 1 < S)
            def _():
                gather(s + 1, obuf, ogs)

            gather_wait(s, buf, gs)

            @pl.when(s >= 2)
            def _():
                writes_wait(s - 2, tb, ws)

            transpose(buf, tb)
            writes(s, tb, ws)

        def body(i, carry):
            s = 2 * i
            step(s, buf_a, tb_a, gs_a, ws_a, buf_b, gs_b)
            step(s + 1, buf_b, tb_b, gs_b, ws_b, buf_a, gs_a)
            return carry

        lax.fori_loop(0, S // 2, body, 0)
        writes_wait(S - 2, tb_a, ws_a)
        writes_wait(S - 1, tb_b, ws_b)

    return k


def kernel(x, table):
    B, S = x.shape
    V, D = table.shape
    info = plsc.get_sparse_core_info()
    NC, NS = info.num_cores, info.num_subcores
    NW = NC * NS
    C = B // _LANES
    R = D // 8
    xw = x.astype(jnp.int32).T.reshape(S, C, _LANES).transpose(1, 0, 2)
    out5 = _build(B, S, D, NC, NS)(xw, table)
    return out5.transpose(2, 4, 0, 1, 3).reshape(B, S, D)
